# Initial kernel scaffold; baseline (speedup 1.0000x reference)
import jax
import jax.numpy as jnp
from jax.experimental import pallas as pl


def kernel(node_s, dist, rbf, r_hat, access_mask, params):
    raise NotImplementedError("write your pallas kernel here")


# speedup vs baseline: 1.2842x; 1.2842x over previous
import jax
import jax.numpy as jnp
from jax.experimental import pallas as pl

B, N, D, R, K, ORDER, H = 2, 96, 64, 32, 8, 3, 64
BI = 8
NEG = -1e30


def _silu(x):
    return x * jax.nn.sigmoid(x)


def _fused_kernel(node_ref, md_ref, mask_ref, rbf_ref, rx_ref, ry_ref, rz_ref,
                  ep_w1, ep_b1, ep_w2, ep_b2,
                  tp_w1, tp_b1, tp_w2, tp_b2,
                  ts_w1, ts_b1, ts_w2t, ts_b2,
                  tm_g, tm_b, tm_w1, tm_b1, tm_w2, tm_b2,
                  eg_w, eg_b, no_g, no_b, no_w, no_b2, en_g, en_b,
                  nd_out, es_out):
    f32 = jnp.float32
    i_blk = pl.program_id(1)
    node_b = node_ref[0]
    md = md_ref[0]
    maskf = mask_ref[0]
    rbf = rbf_ref[0]
    rx = rx_ref[0]
    ry = ry_ref[0]
    rz = rz_ref[0]

    iota_n = jax.lax.broadcasted_iota(jnp.int32, (BI, N), 1)
    vals = md
    idx_cols = []
    for _ in range(K):
        m = jnp.min(vals, axis=1, keepdims=True)
        cand = jnp.where(vals == m, iota_n, N)
        ik = jnp.min(cand, axis=1, keepdims=True)
        idx_cols.append(ik)
        vals = jnp.where(iota_n == ik, 1e9, vals)
    idx = jnp.concatenate(idx_cols, axis=1)

    iota_kn = jax.lax.broadcasted_iota(jnp.int32, (BI, K, N), 2)
    onehot = (idx[:, :, None] == iota_kn).astype(f32)

    tmask = jnp.sum(onehot * maskf[:, None, :], axis=2)
    crx = jnp.sum(onehot * rx[:, None, :], axis=2)
    cry = jnp.sum(onehot * ry[:, None, :], axis=2)
    crz = jnp.sum(onehot * rz[:, None, :], axis=2)

    onehot2 = onehot.reshape(BI * K, N)
    nodecat = jnp.concatenate([node_b, node_b @ ts_w1[D:, :]], axis=1)
    g = onehot2 @ nodecat
    comp_feat = g[:, :D].reshape(BI, K, D)
    cfW = g[:, D:].reshape(BI, K, H)

    rbfW2 = rbf.reshape(BI * N, R) @ tp_w1[ORDER + 1:, :]
    row_a = jax.lax.broadcasted_iota(jnp.int32, (BI, K), 0)
    flatidx = row_a * N + idx
    iota_f = jax.lax.broadcasted_iota(jnp.int32, (BI, K, BI * N), 2)
    onehot_f = (flatidx[:, :, None] == iota_f).astype(f32).reshape(BI * K, BI * N)
    radial_h = (onehot_f @ rbfW2).reshape(BI, K, H)

    cos = (crx[:, :, None] * rx[:, None, :] +
           cry[:, :, None] * ry[:, None, :] +
           crz[:, :, None] * rz[:, None, :])
    cos = jnp.clip(cos, -1.0 + 1e-6, 1.0 - 1e-6)
    p0 = jnp.ones_like(cos)
    p1 = cos
    p2 = (3.0 * cos * p1 - 1.0 * p0) / 2.0
    p3 = (5.0 * cos * p2 - 2.0 * p1) / 3.0

    def _r4(v):
        return v.reshape(1, 1, 1, -1)

    th = (p0[..., None] * _r4(tp_w1[0:1, :])
          + p1[..., None] * _r4(tp_w1[1:2, :])
          + p2[..., None] * _r4(tp_w1[2:3, :])
          + p3[..., None] * _r4(tp_w1[3:4, :])
          + radial_h[:, :, None, :] + _r4(tp_b1[...]))
    th = _silu(th)
    tw = (th.reshape(BI * K * N, H) @ tp_w2[...] + tp_b2[...]).reshape(BI, K, N, D)

    sh = _silu((tw.reshape(BI * K * N, D) @ ts_w1[:D, :]).reshape(BI, K, N, H)
               + cfW[:, :, None, :] + _r4(ts_b1[...]))
    logits = jnp.sum(sh * _r4(ts_w2t[...]), axis=3) + ts_b2[...].reshape(1, 1, 1)

    pm = maskf[:, None, :] * tmask[:, :, None]
    pm = jnp.where(idx[:, :, None] == iota_kn, 0.0, pm)

    logits = jnp.where(pm <= 0.0, NEG, logits)
    lmax = jnp.max(logits, axis=1, keepdims=True)
    e = jnp.exp(logits - lmax)
    attn = e / jnp.sum(e, axis=1, keepdims=True)
    attn = jnp.where(pm > 0.0, attn, 0.0)

    tp_pair = tw * comp_feat[:, :, None, :]
    t_attn = jnp.sum(tp_pair * attn[..., None], axis=1)
    mp = jnp.where(pm[..., None] <= 0.0, NEG, tp_pair)
    t_max = jnp.max(mp, axis=1)
    t_max = jnp.where(t_max <= NEG * 0.5, 0.0, t_max)

    mi = jnp.concatenate([t_attn, t_max], axis=2)
    mu = jnp.mean(mi, axis=2, keepdims=True)
    mv = jnp.mean((mi - mu) ** 2, axis=2, keepdims=True)
    mi = (mi - mu) / jnp.sqrt(mv + 1e-5) * tm_g[...].reshape(1, 1, 2 * D) \
        + tm_b[...].reshape(1, 1, 2 * D)
    mh = _silu(mi.reshape(BI * N, 2 * D) @ tm_w1[...] + tm_b1[...])
    ctx = mh @ tm_w2[...] + tm_b2[...]

    node_i = node_ref[0, pl.ds(i_blk * BI, BI), :]
    hi = node_i @ ep_w1[:D, :]
    dstW = node_b @ ep_w1[D:2 * D, :]
    rbfW1 = (rbf.reshape(BI * N, R) @ ep_w1[2 * D:, :]).reshape(BI, N, H)
    eh = _silu(hi[:, None, :] + dstW[None, :, :] + rbfW1
               + ep_b1[...].reshape(1, 1, H))
    eb = (eh.reshape(BI * N, H) @ ep_w2[...] + ep_b2[...]).reshape(BI, N, D)
    mask3 = maskf[:, :, None]
    eb = eb * mask3

    ef = eb + ctx.reshape(BI, N, D)
    emu = jnp.mean(ef, axis=2, keepdims=True)
    ev = jnp.mean((ef - emu) ** 2, axis=2, keepdims=True)
    ef = (ef - emu) / jnp.sqrt(ev + 1e-5) * en_g[...].reshape(1, 1, D) \
        + en_b[...].reshape(1, 1, D)
    gate = jax.nn.sigmoid(ef.reshape(BI * N, D) @ eg_w[...]
                          + eg_b[...]).reshape(BI, N, D)
    ef = gate * ef

    ns = jnp.sum(ef * mask3, axis=1)
    es = jnp.sum(ef, axis=1)

    nmu = jnp.mean(ns, axis=1, keepdims=True)
    nv = jnp.mean((ns - nmu) ** 2, axis=1, keepdims=True)
    nd = (ns - nmu) / jnp.sqrt(nv + 1e-5) * no_g[...] + no_b[...]
    nd = nd @ no_w[...] + no_b2[...]

    nd_out[0] = nd
    es_out[0] = es


def kernel(node_s, dist, rbf, r_hat, access_mask, params):
    p = params
    f32 = jnp.float32
    maskf = access_mask.astype(f32)
    maxd = jnp.maximum(dist.max(axis=(1, 2), keepdims=True), 1.0) + 1.0
    md = jnp.where(access_mask, dist, maxd)
    rx = r_hat[..., 0]
    ry = r_hat[..., 1]
    rz = r_hat[..., 2]

    def row2(v):
        return v.reshape(1, -1)

    args = (node_s, md, maskf, rbf, rx, ry, rz,
            p['ep_w1'], row2(p['ep_b1']), p['ep_w2'], row2(p['ep_b2']),
            p['tp_w1'], row2(p['tp_b1']), p['tp_w2'], row2(p['tp_b2']),
            p['ts_w1'], row2(p['ts_b1']), p['ts_w2'].T, row2(p['ts_b2']),
            row2(p['tm_g']), row2(p['tm_b']),
            p['tm_w1'], row2(p['tm_b1']), p['tm_w2'], row2(p['tm_b2']),
            p['eg_w'], row2(p['eg_b']), row2(p['no_g']), row2(p['no_b']),
            p['no_w'], row2(p['no_b2']), row2(p['en_g']), row2(p['en_b']))

    def full(a):
        return pl.BlockSpec(a.shape, lambda b, i: (0,) * a.ndim)

    row_specs = [
        pl.BlockSpec((1, N, D), lambda b, i: (b, 0, 0)),
        pl.BlockSpec((1, BI, N), lambda b, i: (b, i, 0)),
        pl.BlockSpec((1, BI, N), lambda b, i: (b, i, 0)),
        pl.BlockSpec((1, BI, N, R), lambda b, i: (b, i, 0, 0)),
        pl.BlockSpec((1, BI, N), lambda b, i: (b, i, 0)),
        pl.BlockSpec((1, BI, N), lambda b, i: (b, i, 0)),
        pl.BlockSpec((1, BI, N), lambda b, i: (b, i, 0)),
    ]
    in_specs = row_specs + [full(a) for a in args[7:]]

    nd, es = pl.pallas_call(
        _fused_kernel,
        grid=(B, N // BI),
        in_specs=in_specs,
        out_specs=[
            pl.BlockSpec((1, BI, D), lambda b, i: (b, i, 0)),
            pl.BlockSpec((1, BI, D), lambda b, i: (b, i, 0)),
        ],
        out_shape=[
            jax.ShapeDtypeStruct((B, N, D), f32),
            jax.ShapeDtypeStruct((B, N, D), f32),
        ],
    )(*args)

    denom = jnp.maximum(maskf.sum(axis=(1, 2)), 1.0)[:, None]
    bond_graph = es.sum(axis=1) / denom
    return nd, bond_graph


# speedup vs baseline: 1.6008x; 1.2466x over previous
import jax
import jax.numpy as jnp
from jax.experimental import pallas as pl

B, N, D, R, K, ORDER, H = 2, 96, 64, 32, 8, 3, 64
BI = 16
NEG = -1e30


def _silu(x):
    return x * jax.nn.sigmoid(x)


def _fused_kernel(node_ref, md_ref, mask_ref, rbf_ref, rx_ref, ry_ref, rz_ref,
                  ep_w1, ep_b1, ep_w2, ep_b2,
                  tp_w1, tp_c, tp_b1, tp_w2, tp_b2,
                  ts_w1, ts_b1, ts_w2t, ts_b2,
                  tm_g, tm_b, tm_w1, tm_b1, tm_w2, tm_b2,
                  eg_w, eg_b, no_g, no_b, no_w, no_b2, en_g, en_b,
                  nd_out, es_out):
    f32 = jnp.float32
    i_blk = pl.program_id(1)
    node_b = node_ref[0]
    md = md_ref[0]
    maskf = mask_ref[0]
    rbf = rbf_ref[0]
    rx = rx_ref[0]
    ry = ry_ref[0]
    rz = rz_ref[0]

    iota_n = jax.lax.broadcasted_iota(jnp.int32, (BI, N), 1)
    vals = md
    idx_cols = []
    for _ in range(K):
        m = jnp.min(vals, axis=1, keepdims=True)
        cand = jnp.where(vals == m, iota_n, N)
        ik = jnp.min(cand, axis=1, keepdims=True)
        idx_cols.append(ik)
        vals = jnp.where(iota_n == ik, 1e9, vals)
    idx = jnp.concatenate(idx_cols, axis=1)

    iota_kn = jax.lax.broadcasted_iota(jnp.int32, (BI, K, N), 2)
    onehot = (idx[:, :, None] == iota_kn).astype(f32)

    tmask = jnp.sum(onehot * maskf[:, None, :], axis=2)
    crx = jnp.sum(onehot * rx[:, None, :], axis=2)
    cry = jnp.sum(onehot * ry[:, None, :], axis=2)
    crz = jnp.sum(onehot * rz[:, None, :], axis=2)

    onehot2 = onehot.reshape(BI * K, N)
    nodecat = jnp.concatenate([node_b, node_b @ ts_w1[D:, :]], axis=1)
    g = onehot2 @ nodecat
    comp_feat = g[:, :D].reshape(BI, K, D)
    cfW = g[:, D:].reshape(BI, K, H)

    rbfW2 = rbf.reshape(BI * N, R) @ tp_w1[ORDER + 1:, :]
    row_a = jax.lax.broadcasted_iota(jnp.int32, (BI, K), 0)
    flatidx = row_a * N + idx
    iota_f = jax.lax.broadcasted_iota(jnp.int32, (BI, K, BI * N), 2)
    onehot_f = (flatidx[:, :, None] == iota_f).astype(f32).reshape(BI * K, BI * N)
    radial_h = (onehot_f @ rbfW2).reshape(BI, K, H)

    cos = (crx[:, :, None] * rx[:, None, :] +
           cry[:, :, None] * ry[:, None, :] +
           crz[:, :, None] * rz[:, None, :])
    cos = jnp.clip(cos, -1.0 + 1e-6, 1.0 - 1e-6)

    def _r4(v):
        return v.reshape(1, 1, 1, -1)

    base = radial_h + tp_c[0:1, :][None, :, :] + tp_b1[...][None, :, :]
    x = cos[..., None]
    th = (x * _r4(tp_c[3:4, :]) + _r4(tp_c[2:3, :])) * x
    th = (th + _r4(tp_c[1:2, :])) * x + base[:, :, None, :]
    th = _silu(th)
    tw = (th.reshape(BI * K * N, H) @ tp_w2[...] + tp_b2[...]).reshape(BI, K, N, D)

    sh = _silu((tw.reshape(BI * K * N, D) @ ts_w1[:D, :]).reshape(BI, K, N, H)
               + cfW[:, :, None, :] + _r4(ts_b1[...]))
    logits = jnp.sum(sh * _r4(ts_w2t[...]), axis=3) + ts_b2[...].reshape(1, 1, 1)

    pm = maskf[:, None, :] * tmask[:, :, None]
    pm = jnp.where(idx[:, :, None] == iota_kn, 0.0, pm)

    logits = jnp.where(pm <= 0.0, NEG, logits)
    lmax = jnp.max(logits, axis=1, keepdims=True)
    e = jnp.exp(logits - lmax)
    attn = e / jnp.sum(e, axis=1, keepdims=True)
    attn = jnp.where(pm > 0.0, attn, 0.0)

    tp_pair = tw * comp_feat[:, :, None, :]
    t_attn = jnp.sum(tp_pair * attn[..., None], axis=1)
    mp = jnp.where(pm[..., None] <= 0.0, NEG, tp_pair)
    t_max = jnp.max(mp, axis=1)
    t_max = jnp.where(t_max <= NEG * 0.5, 0.0, t_max)

    mi = jnp.concatenate([t_attn, t_max], axis=2)
    mu = jnp.mean(mi, axis=2, keepdims=True)
    mv = jnp.mean((mi - mu) ** 2, axis=2, keepdims=True)
    mi = (mi - mu) * (1.0 / jnp.sqrt(mv + 1e-5)) * tm_g[...].reshape(1, 1, 2 * D) \
        + tm_b[...].reshape(1, 1, 2 * D)
    mh = _silu(mi.reshape(BI * N, 2 * D) @ tm_w1[...] + tm_b1[...])
    ctx = mh @ tm_w2[...] + tm_b2[...]

    node_i = node_ref[0, pl.ds(i_blk * BI, BI), :]
    hi = node_i @ ep_w1[:D, :]
    dstW = node_b @ ep_w1[D:2 * D, :]
    rbfW1 = (rbf.reshape(BI * N, R) @ ep_w1[2 * D:, :]).reshape(BI, N, H)
    eh = _silu(hi[:, None, :] + dstW[None, :, :] + rbfW1
               + ep_b1[...].reshape(1, 1, H))
    eb = (eh.reshape(BI * N, H) @ ep_w2[...] + ep_b2[...]).reshape(BI, N, D)
    mask3 = maskf[:, :, None]
    eb = eb * mask3

    ef = eb + ctx.reshape(BI, N, D)
    emu = jnp.mean(ef, axis=2, keepdims=True)
    ev = jnp.mean((ef - emu) ** 2, axis=2, keepdims=True)
    ef = (ef - emu) * (1.0 / jnp.sqrt(ev + 1e-5)) * en_g[...].reshape(1, 1, D) \
        + en_b[...].reshape(1, 1, D)
    gate = jax.nn.sigmoid(ef.reshape(BI * N, D) @ eg_w[...]
                          + eg_b[...]).reshape(BI, N, D)
    ef = gate * ef

    ns = jnp.sum(ef * mask3, axis=1)
    es = jnp.sum(ef, axis=1)

    nmu = jnp.mean(ns, axis=1, keepdims=True)
    nv = jnp.mean((ns - nmu) ** 2, axis=1, keepdims=True)
    nd = (ns - nmu) * (1.0 / jnp.sqrt(nv + 1e-5)) * no_g[...] + no_b[...]
    nd = nd @ no_w[...] + no_b2[...]

    nd_out[0] = nd
    es_out[0] = es


def kernel(node_s, dist, rbf, r_hat, access_mask, params):
    p = params
    f32 = jnp.float32
    maskf = access_mask.astype(f32)
    maxd = jnp.maximum(dist.max(axis=(1, 2), keepdims=True), 1.0) + 1.0
    md = jnp.where(access_mask, dist, maxd)
    rx = r_hat[..., 0]
    ry = r_hat[..., 1]
    rz = r_hat[..., 2]

    def row2(v):
        return v.reshape(1, -1)

    w = p['tp_w1']
    tp_c = jnp.stack([w[0] - 0.5 * w[2], w[1] - 1.5 * w[3],
                      1.5 * w[2], 2.5 * w[3]], axis=0)

    args = (node_s, md, maskf, rbf, rx, ry, rz,
            p['ep_w1'], row2(p['ep_b1']), p['ep_w2'], row2(p['ep_b2']),
            p['tp_w1'], tp_c, row2(p['tp_b1']), p['tp_w2'], row2(p['tp_b2']),
            p['ts_w1'], row2(p['ts_b1']), p['ts_w2'].T, row2(p['ts_b2']),
            row2(p['tm_g']), row2(p['tm_b']),
            p['tm_w1'], row2(p['tm_b1']), p['tm_w2'], row2(p['tm_b2']),
            p['eg_w'], row2(p['eg_b']), row2(p['no_g']), row2(p['no_b']),
            p['no_w'], row2(p['no_b2']), row2(p['en_g']), row2(p['en_b']))

    def full(a):
        return pl.BlockSpec(a.shape, lambda b, i: (0,) * a.ndim)

    row_specs = [
        pl.BlockSpec((1, N, D), lambda b, i: (b, 0, 0)),
        pl.BlockSpec((1, BI, N), lambda b, i: (b, i, 0)),
        pl.BlockSpec((1, BI, N), lambda b, i: (b, i, 0)),
        pl.BlockSpec((1, BI, N, R), lambda b, i: (b, i, 0, 0)),
        pl.BlockSpec((1, BI, N), lambda b, i: (b, i, 0)),
        pl.BlockSpec((1, BI, N), lambda b, i: (b, i, 0)),
        pl.BlockSpec((1, BI, N), lambda b, i: (b, i, 0)),
    ]
    in_specs = row_specs + [full(a) for a in args[7:]]

    nd, es = pl.pallas_call(
        _fused_kernel,
        grid=(B, N // BI),
        in_specs=in_specs,
        out_specs=[
            pl.BlockSpec((1, BI, D), lambda b, i: (b, i, 0)),
            pl.BlockSpec((1, BI, D), lambda b, i: (b, i, 0)),
        ],
        out_shape=[
            jax.ShapeDtypeStruct((B, N, D), f32),
            jax.ShapeDtypeStruct((B, N, D), f32),
        ],
    )(*args)

    denom = jnp.maximum(maskf.sum(axis=(1, 2)), 1.0)[:, None]
    bond_graph = es.sum(axis=1) / denom
    return nd, bond_graph
